# baseline (speedup 1.0000x reference)
import functools

import jax
import jax.numpy as jnp
from jax import lax
from jax.experimental import pallas as pl
from jax.experimental.pallas import tpu as pltpu
from jax.experimental.pallas import tpu_sc as plsc

_CH = 512


def _sc_filter(x_hbm, o_hbm, zbuf, sem):
    wid = lax.axis_index("s") * 2 + lax.axis_index("c")

    def zinit(i, _):
        r = lax.div(i, 24 * 16)
        rr = lax.rem(i, 24 * 16)
        h = lax.div(rr, 16)
        l = lax.rem(rr, 16)
        zbuf[r, h, pl.ds(l * 16, 16)] = jnp.zeros((16,), jnp.float32)
        return 0

    lax.fori_loop(0, 12 * 24 * 16, zinit, 0)

    for j in range(2):
        b = wid * 2 + j
        pltpu.make_async_copy(
            x_hbm.at[b, :, :, pl.ds(0, _CH)],
            o_hbm.at[b, :, :, pl.ds(0, _CH)],
            sem).start()
        pltpu.make_async_copy(
            zbuf, o_hbm.at[b, pl.ds(0, 12), :, pl.ds(_CH, 256)], sem).start()
        pltpu.make_async_copy(
            zbuf, o_hbm.at[b, pl.ds(12, 12), :, pl.ds(_CH, 256)], sem).start()
    for j in range(2):
        b = wid * 2 + j
        pltpu.make_async_copy(
            x_hbm.at[b, :, :, pl.ds(0, _CH)],
            o_hbm.at[b, :, :, pl.ds(0, _CH)],
            sem).wait()
        pltpu.make_async_copy(
            zbuf, o_hbm.at[b, pl.ds(0, 12), :, pl.ds(_CH, 256)], sem).wait()
        pltpu.make_async_copy(
            zbuf, o_hbm.at[b, pl.ds(12, 12), :, pl.ds(_CH, 256)], sem).wait()


def kernel(x, channels):
    B, C, H, W = x.shape
    xt = jnp.transpose(x, (0, 2, 3, 1))
    mesh = plsc.VectorSubcoreMesh(core_axis_name="c", subcore_axis_name="s")
    k = functools.partial(
        pl.kernel, mesh=mesh,
        out_type=jax.ShapeDtypeStruct((B, H, W, C), x.dtype),
        scratch_types=[
            pltpu.VMEM((12, 24, 256), jnp.float32),
            pltpu.SemaphoreType.DMA,
        ],
    )(_sc_filter)
    out = k(xt)
    return jnp.transpose(out, (0, 3, 1, 2))


# speedup vs baseline: 39.7499x; 39.7499x over previous
import functools

import jax
import jax.numpy as jnp
from jax.experimental import pallas as pl
from jax.experimental.pallas import tpu as pltpu

_CUTS = (0, 512, 640, 768)
_BB = 8
_SLOTS = 2


def _filter_kernel(B, C, H, W, ch_ref, x_ref, o_ref, xbuf, isem):
    NK = len(_CUTS) - 1
    NB = B // _BB
    ch = ch_ref[0]
    i = pl.program_id(0)

    def chunk_copy(ii, slot, k):
        lo, hi = _CUTS[k], _CUTS[k + 1]
        return pltpu.make_async_copy(
            x_ref.at[pl.ds(ii * _BB, _BB), :, :, pl.ds(lo, hi - lo)],
            xbuf.at[slot, :, :, :, pl.ds(lo, hi - lo)],
            isem.at[slot, k])

    def start_block(ii):
        for k in range(NK):
            @pl.when(_CUTS[k] < ch)
            def _(k=k):
                chunk_copy(ii, jax.lax.rem(ii, _SLOTS), k).start()

    @pl.when(i == 0)
    def _():
        for j in range(_SLOTS - 1):
            start_block(jnp.int32(j))

    @pl.when(i + _SLOTS - 1 < NB)
    def _():
        start_block(i + _SLOTS - 1)

    slot = jax.lax.rem(i, _SLOTS)
    for k in range(NK):
        @pl.when(_CUTS[k] < ch)
        def _(k=k):
            chunk_copy(i, slot, k).wait()

    c = jax.lax.broadcasted_iota(jnp.int32, (_BB, H, W, C), 3)
    o_ref[...] = jnp.where(c < ch, xbuf[slot], 0.0)


def kernel(x, channels):
    B, C, H, W = x.shape
    xt = jnp.transpose(x, (0, 2, 3, 1))
    ch = jnp.asarray(channels, jnp.int32).reshape(1)
    out = pl.pallas_call(
        functools.partial(_filter_kernel, B, C, H, W),
        grid_spec=pltpu.PrefetchScalarGridSpec(
            num_scalar_prefetch=1,
            grid=(B // _BB,),
            in_specs=[pl.BlockSpec(memory_space=pltpu.MemorySpace.HBM)],
            out_specs=pl.BlockSpec((_BB, H, W, C), lambda i, ch: (i, 0, 0, 0)),
            scratch_shapes=[
                pltpu.VMEM((_SLOTS, _BB, H, W, C), x.dtype),
                pltpu.SemaphoreType.DMA((_SLOTS, len(_CUTS) - 1)),
            ],
        ),
        out_shape=jax.ShapeDtypeStruct((B, H, W, C), x.dtype),
    )(ch, xt)
    return jnp.transpose(out, (0, 3, 1, 2))
